# baseline (speedup 1.0000x reference)
import jax
import jax.numpy as jnp
from jax.experimental import pallas as pl
from jax.experimental.pallas import tpu as pltpu
from jax.experimental.pallas import tpu_sc as plsc


_LANES = 16
_BR = 16


def _sc_pos_add(x2d, pos_emb, t):
    n, c = x2d.shape
    nblk_t = t // _BR
    mesh = plsc.VectorSubcoreMesh(core_axis_name="c", subcore_axis_name="s")

    @pl.kernel(out_type=jax.ShapeDtypeStruct((n, c), x2d.dtype), mesh=mesh)
    def run(x_hbm, pe_hbm, o_hbm):
        def body(x_vmem, pe_vmem, o_vmem):
            @pl.loop(0, _BR)
            def _(r):
                @pl.loop(0, c, step=_LANES, unroll=8)
                def _(cc):
                    slc = (pl.ds(r, 1), pl.ds(cc, _LANES))
                    o_vmem.at[*slc][...] = (
                        x_vmem.at[*slc][...] + pe_vmem.at[*slc][...]
                    )

        pltpu.emit_pipeline(
            body,
            grid=(n // _BR,),
            in_specs=[
                pl.BlockSpec((_BR, c), lambda i: (i, 0)),
                pl.BlockSpec((_BR, c), lambda i: (jax.lax.rem(i, nblk_t), 0)),
            ],
            out_specs=[pl.BlockSpec((_BR, c), lambda i: (i, 0))],
            core_axis_name=("c", "s"),
            dimension_semantics=(pltpu.PARALLEL,),
        )(x_hbm, pe_hbm, o_hbm)

    return run(x2d, pos_emb)


def kernel(x, pos_emb):
    b, t, c = x.shape
    out2d = _sc_pos_add(x.reshape(b * t, c), pos_emb, t)
    return out2d.reshape(b, t, c)


# speedup vs baseline: 4.1942x; 4.1942x over previous
import jax
import jax.numpy as jnp
from jax.experimental import pallas as pl


def _add_block(x_ref, pe_ref, o_ref):
    o_ref[...] = x_ref[...] + pe_ref[...]


def kernel(x, pos_emb):
    b, t, c = x.shape
    bt = 2048
    grid = (t // bt, b)
    return pl.pallas_call(
        _add_block,
        grid=grid,
        in_specs=[
            pl.BlockSpec((1, bt, c), lambda i, j: (j, i, 0)),
            pl.BlockSpec((bt, c), lambda i, j: (i, 0)),
        ],
        out_specs=pl.BlockSpec((1, bt, c), lambda i, j: (j, i, 0)),
        out_shape=jax.ShapeDtypeStruct((b, t, c), x.dtype),
    )(x, pos_emb)
